# Initial kernel scaffold; baseline (speedup 1.0000x reference)
import jax
import jax.numpy as jnp
from jax.experimental import pallas as pl


def kernel(words_seq, table):
    raise NotImplementedError("write your pallas kernel here")


# speedup vs baseline: 1.4588x; 1.4588x over previous
import functools

import jax
import jax.numpy as jnp
from jax import lax
from jax.experimental import pallas as pl
from jax.experimental.pallas import tpu as pltpu
from jax.experimental.pallas import tpu_sc as plsc

VOCAB = 1000000
EMBED = 32
BATCH = 4096
SEQ = 200

ROW = 128
NROWS = BATCH * SEQ // ROW
NW = 32
ROWS_PER_W = NROWS // NW
K = 8
NGROUPS = ROWS_PER_W // K


def _make_kernel():
  mesh = plsc.VectorSubcoreMesh(core_axis_name="c", subcore_axis_name="s")

  @functools.partial(
      pl.kernel,
      mesh=mesh,
      compiler_params=pltpu.CompilerParams(use_tc_tiling_on_sc=False),
      out_type=jax.ShapeDtypeStruct((NROWS, ROW, EMBED), jnp.float32),
      scratch_types=[
          pltpu.VMEM((K, ROW), jnp.int32),
          pltpu.VMEM((K, ROW, EMBED), jnp.float32),
          pltpu.SemaphoreType.DMA,
      ],
  )
  def body(idx_hbm, table_hbm, out_hbm, idx_v, rows_v, sem):
    wid = lax.axis_index("s") * 2 + lax.axis_index("c")
    base = wid * ROWS_PER_W

    def group(g, carry):
      row = base + g * K
      pltpu.sync_copy(idx_hbm.at[pl.ds(row, K)], idx_v)
      copies = [
          pltpu.async_copy(table_hbm.at[idx_v.at[j]], rows_v.at[j], sem)
          for j in range(K)
      ]
      for c in copies:
        c.wait()
      pltpu.sync_copy(rows_v, out_hbm.at[pl.ds(row, K)])
      return carry

    lax.fori_loop(0, NGROUPS, group, 0)

  return body


_sc_gather = _make_kernel()


def kernel(words_seq, table):
  idx = words_seq.reshape(NROWS, ROW).astype(jnp.int32)
  out = _sc_gather(idx, table)
  return out.reshape(BATCH, SEQ, EMBED)


# speedup vs baseline: 1.4949x; 1.0247x over previous
import functools

import jax
import jax.numpy as jnp
from jax import lax
from jax.experimental import pallas as pl
from jax.experimental.pallas import tpu as pltpu
from jax.experimental.pallas import tpu_sc as plsc

VOCAB = 1000000
EMBED = 32
BATCH = 4096
SEQ = 200

ROW = 128
NROWS = BATCH * SEQ // ROW
NW = 32
ROWS_PER_W = NROWS // NW
K = 10
NGROUPS = ROWS_PER_W // K


def _make_kernel():
  mesh = plsc.VectorSubcoreMesh(core_axis_name="c", subcore_axis_name="s")

  @functools.partial(
      pl.kernel,
      mesh=mesh,
      compiler_params=pltpu.CompilerParams(use_tc_tiling_on_sc=False),
      out_type=jax.ShapeDtypeStruct((NROWS, ROW, EMBED), jnp.float32),
      scratch_types=[
          pltpu.VMEM((ROWS_PER_W, ROW), jnp.int32),
          pltpu.VMEM((K, ROW, EMBED), jnp.float32),
          pltpu.VMEM((K, ROW, EMBED), jnp.float32),
          pltpu.SemaphoreType.DMA,
          pltpu.SemaphoreType.DMA,
          pltpu.SemaphoreType.DMA,
          pltpu.SemaphoreType.DMA,
      ],
  )
  def body(idx_hbm, table_hbm, out_hbm, idx_all, rows0, rows1,
           gsem0, gsem1, wsem0, wsem1):
    wid = lax.axis_index("s") * 2 + lax.axis_index("c")
    base = wid * ROWS_PER_W

    pltpu.sync_copy(idx_hbm.at[pl.ds(base, ROWS_PER_W)], idx_all)

    def fire(g, rows, gsem):
      for j in range(K):
        pltpu.async_copy(table_hbm.at[idx_all.at[g * K + j]], rows.at[j], gsem)

    def drain(g, rows, gsem):
      pltpu.make_async_copy(out_hbm.at[pl.ds(0, K)], rows, gsem).wait()

    def wb(g, rows, wsem):
      pltpu.async_copy(rows, out_hbm.at[pl.ds(base + g * K, K)], wsem)

    def wait_wb(g, rows, wsem):
      pltpu.make_async_copy(rows, out_hbm.at[pl.ds(base + g * K, K)], wsem).wait()

    fire(0, rows0, gsem0)
    fire(1, rows1, gsem1)
    drain(0, rows0, gsem0)
    wb(0, rows0, wsem0)

    def step(m, carry):
      g0 = 2 * m
      g1 = 2 * m + 1
      drain(g1 - 2, rows1, gsem1)
      wb(g1 - 2, rows1, wsem1)
      wait_wb(g0 - 2, rows0, wsem0)
      fire(g0, rows0, gsem0)
      drain(g0, rows0, gsem0)
      wb(g0, rows0, wsem0)
      wait_wb(g1 - 2, rows1, wsem1)
      fire(g1, rows1, gsem1)
      return carry

    lax.fori_loop(1, NGROUPS // 2, step, 0)

    drain(NGROUPS - 1, rows1, gsem1)
    wb(NGROUPS - 1, rows1, wsem1)
    wait_wb(NGROUPS - 2, rows0, wsem0)
    wait_wb(NGROUPS - 1, rows1, wsem1)

  return body


_sc_gather = _make_kernel()


def kernel(words_seq, table):
  idx = words_seq.reshape(NROWS, ROW).astype(jnp.int32)
  out = _sc_gather(idx, table)
  return out.reshape(BATCH, SEQ, EMBED)
